# Initial kernel scaffold; baseline (speedup 1.0000x reference)
import jax
import jax.numpy as jnp
from jax.experimental import pallas as pl


def kernel(nodes_feat, nodes_num_norm_sqrt, edges_feat, edges_num_norm_sqrt, W_embed, b_embed, Ws, bs, gammas, betas, W_r0, b_r0, W_r1, b_r1, W_r2, b_r2, edge_index, graph_ids):
    raise NotImplementedError("write your pallas kernel here")


# speedup vs baseline: 3.5040x; 3.5040x over previous
import functools

import jax
import jax.numpy as jnp
from jax import lax
from jax.experimental import pallas as pl
from jax.experimental.pallas import tpu as pltpu
from jax.experimental.pallas import tpu_sc as plsc

NN = 10000
EE = 320000
DD = 128
HH = 128
GG = 128
LL = 4
NCLS = 10

SC_CORES = 2
SC_TILES = 16
HALF = HH // 2
CHUNK = 128
NCH = 2560
EPAD = NCH * CHUNK
CMAX = NCH // SC_TILES
CMAXD = NCH // (SC_CORES * SC_TILES)
NN_PAD = 10240
RPT = NN_PAD // SC_TILES
DEGW = 16

_sc_mesh = plsc.VectorSubcoreMesh(core_axis_name="c", subcore_axis_name="s")
_sc_params = pltpu.CompilerParams(use_tc_tiling_on_sc=False)


def _zero_rows(ref, nrows, ncols):
    zer = jnp.zeros((16,), jnp.float32)

    def body(i, _):
        for k in range(ncols // 16):
            ref[i, pl.ds(k * 16, 16)] = zer
        return 0

    lax.fori_loop(0, nrows, body, 0)


def _fill_ones_rows(ref, nrows, ncols):
    one = jnp.ones((16,), jnp.float32)

    def body(i, _):
        for k in range(ncols // 16):
            ref[i, pl.ds(k * 16, 16)] = one
        return 0

    lax.fori_loop(0, nrows, body, 0)


@functools.partial(
    pl.kernel,
    out_type=jax.ShapeDtypeStruct((SC_CORES, 2, NN_PAD, DEGW), jnp.float32),
    mesh=_sc_mesh,
    compiler_params=_sc_params,
    scratch_types=[
        pltpu.VMEM_SHARED((NN_PAD, DEGW), jnp.float32),
        pltpu.VMEM_SHARED((NN_PAD, DEGW), jnp.float32),
        pltpu.VMEM((CMAXD, CHUNK), jnp.int32),
        pltpu.VMEM((CMAXD, CHUNK), jnp.int32),
        pltpu.VMEM((CHUNK, DEGW), jnp.float32),
        pltpu.VMEM((RPT, DEGW), jnp.float32),
    ],
)
def _deg_kernel(srcm, dstm, out, dout_sh, din_sh, src_v, dst_v, ones_v, zer_v):
    c = lax.axis_index("c")
    s = lax.axis_index("s")
    w = c * SC_TILES + s

    _fill_ones_rows(ones_v, CHUNK, DEGW)
    _zero_rows(zer_v, RPT, DEGW)
    pltpu.sync_copy(zer_v, dout_sh.at[pl.ds(s * RPT, RPT), :])
    pltpu.sync_copy(zer_v, din_sh.at[pl.ds(s * RPT, RPT), :])
    plsc.subcore_barrier()

    lo = w * CMAXD
    pltpu.sync_copy(srcm.at[pl.ds(lo, CMAXD), :], src_v)
    pltpu.sync_copy(dstm.at[pl.ds(lo, CMAXD), :], dst_v)

    def body(j, _):
        pltpu.sync_copy(ones_v, dout_sh.at[src_v.at[j]], add=True)
        pltpu.sync_copy(ones_v, din_sh.at[dst_v.at[j]], add=True)
        return 0

    lax.fori_loop(0, CMAXD, body, 0)
    plsc.subcore_barrier()

    pltpu.sync_copy(dout_sh.at[pl.ds(s * RPT, RPT), :],
                    out.at[c, 0, pl.ds(s * RPT, RPT), :])
    pltpu.sync_copy(din_sh.at[pl.ds(s * RPT, RPT), :],
                    out.at[c, 1, pl.ds(s * RPT, RPT), :])


@functools.partial(
    pl.kernel,
    out_type=jax.ShapeDtypeStruct((SC_CORES, NN_PAD, HALF), jnp.float32),
    mesh=_sc_mesh,
    compiler_params=_sc_params,
    scratch_types=[
        pltpu.VMEM_SHARED((NN_PAD, HALF), jnp.float32),
        pltpu.VMEM((CMAX, CHUNK), jnp.int32),
        pltpu.VMEM((CMAX, CHUNK), jnp.int32),
        pltpu.VMEM((CHUNK, HALF), jnp.float32),
        pltpu.VMEM((RPT, HALF), jnp.float32),
    ],
)
def _mp_kernel(xh, srcm, dstm, aggh, agg_sh, src_v, dst_v, rows_v, zer_v):
    c = lax.axis_index("c")
    s = lax.axis_index("s")

    _zero_rows(zer_v, RPT, HALF)
    pltpu.sync_copy(zer_v, agg_sh.at[pl.ds(s * RPT, RPT), :])
    plsc.subcore_barrier()

    lo = s * CMAX
    pltpu.sync_copy(srcm.at[pl.ds(lo, CMAX), :], src_v)
    pltpu.sync_copy(dstm.at[pl.ds(lo, CMAX), :], dst_v)

    def body(j, _):
        pltpu.sync_copy(xh.at[c].at[src_v.at[j]], rows_v)
        pltpu.sync_copy(rows_v, agg_sh.at[dst_v.at[j]], add=True)
        return 0

    lax.fori_loop(0, CMAX, body, 0)
    plsc.subcore_barrier()

    pltpu.sync_copy(agg_sh.at[pl.ds(s * RPT, RPT), :],
                    aggh.at[c, pl.ds(s * RPT, RPT), :])


def _embed_body(nf, w, b, ns, h_out, x_out):
    h = jnp.dot(nf[...], w[...], preferred_element_type=jnp.float32) + b[...]
    h_out[...] = h
    xs = h * ns[...]
    x_out[0, pl.ds(0, NN), :] = xs[:, :HALF]
    x_out[1, pl.ds(0, NN), :] = xs[:, HALF:]


def _embed_call(nf, w, b, ns):
    return pl.pallas_call(
        _embed_body,
        out_shape=[
            jax.ShapeDtypeStruct((NN, HH), jnp.float32),
            jax.ShapeDtypeStruct((SC_CORES, NN_PAD, HALF), jnp.float32),
        ],
    )(nf, w, b, ns)


def _layer_body(agg, h_in, nd, nns, w, b, gamma, beta, ns, h_out, x_out):
    a = agg[...]
    aggf = jnp.concatenate([a[0, :NN], a[1, :NN]], axis=1) * nd[...]
    hc = jnp.dot(aggf, w[...], preferred_element_type=jnp.float32) + b[...]
    hc = hc * nns[...]
    mean = jnp.mean(hc, axis=0, keepdims=True)
    cent = hc - mean
    var = jnp.mean(cent * cent, axis=0, keepdims=True)
    hn = cent * lax.rsqrt(var + 1e-5) * gamma[...] + beta[...]
    h = h_in[...] + jnp.maximum(hn, 0.0)
    h_out[...] = h
    xs = h * ns[...]
    x_out[0, pl.ds(0, NN), :] = xs[:, :HALF]
    x_out[1, pl.ds(0, NN), :] = xs[:, HALF:]


def _layer_call(agg, h_in, nd, nns, w, b, gamma, beta, ns):
    return pl.pallas_call(
        _layer_body,
        out_shape=[
            jax.ShapeDtypeStruct((NN, HH), jnp.float32),
            jax.ShapeDtypeStruct((SC_CORES, NN_PAD, HALF), jnp.float32),
        ],
    )(agg, h_in, nd, nns, w, b, gamma, beta, ns)


def _readout_body(h, gid, w0, b0, w1, b1, w2, b2, out):
    iota = lax.broadcasted_iota(jnp.int32, (1, GG), 1)
    onehot = (gid[...] == iota).astype(jnp.float32)
    dn = (((0,), (0,)), ((), ()))
    hsum = lax.dot_general(onehot, h[...], dn,
                           preferred_element_type=jnp.float32)
    counts = lax.dot_general(onehot, jnp.ones((NN, 1), jnp.float32), dn,
                             preferred_element_type=jnp.float32)
    hg = hsum / jnp.maximum(counts, 1.0)
    y = jnp.maximum(jnp.dot(hg, w0[...], preferred_element_type=jnp.float32)
                    + b0[...], 0.0)
    y = jnp.maximum(jnp.dot(y, w1[...], preferred_element_type=jnp.float32)
                    + b1[...], 0.0)
    out[...] = jnp.dot(y, w2[...], preferred_element_type=jnp.float32) + b2[...]


def _readout_call(h, gid, w0, b0, w1, b1, w2, b2):
    return pl.pallas_call(
        _readout_body,
        out_shape=jax.ShapeDtypeStruct((GG, NCLS), jnp.float32),
    )(h, gid, w0, b0, w1, b1, w2, b2)


def kernel(nodes_feat, nodes_num_norm_sqrt, edges_feat, edges_num_norm_sqrt,
           W_embed, b_embed, Ws, bs, gammas, betas,
           W_r0, b_r0, W_r1, b_r1, W_r2, b_r2,
           edge_index, graph_ids):
    pad = jnp.full((2, EPAD - EE), NN, dtype=jnp.int32)
    ei = jnp.concatenate([edge_index, pad], axis=1)
    srcm = ei[0].reshape(NCH, CHUNK)
    dstm = ei[1].reshape(NCH, CHUNK)

    deg = _deg_kernel(srcm, dstm)
    deg_out = deg[0, 0, :NN, 0] + deg[1, 0, :NN, 0]
    deg_in = deg[0, 1, :NN, 0] + deg[1, 1, :NN, 0]
    norm_src = lax.rsqrt(jnp.maximum(deg_out, 1.0)).reshape(NN, 1)
    norm_dst = lax.rsqrt(jnp.maximum(deg_in, 1.0)).reshape(NN, 1)

    h, x = _embed_call(nodes_feat, W_embed, b_embed.reshape(1, HH), norm_src)
    for i in range(LL):
        agg = _mp_kernel(x, srcm, dstm)
        h, x = _layer_call(agg, h, norm_dst, nodes_num_norm_sqrt,
                           Ws[i], bs[i].reshape(1, HH),
                           gammas[i].reshape(1, HH), betas[i].reshape(1, HH),
                           norm_src)

    return _readout_call(h, graph_ids.reshape(NN, 1),
                         W_r0, b_r0.reshape(1, -1),
                         W_r1, b_r1.reshape(1, -1),
                         W_r2, b_r2.reshape(1, -1))


# speedup vs baseline: 4.4583x; 1.2723x over previous
import functools

import jax
import jax.numpy as jnp
from jax import lax
from jax.experimental import pallas as pl
from jax.experimental.pallas import tpu as pltpu
from jax.experimental.pallas import tpu_sc as plsc

NN = 10000
EE = 320000
DD = 128
HH = 128
GG = 128
LL = 4
NCLS = 10

SC_CORES = 2
SC_TILES = 16
HALF = HH // 2
CHUNK = 128
NCH = 2560
EPAD = NCH * CHUNK
CMAX = NCH // SC_TILES
CMAXD = NCH // (SC_CORES * SC_TILES)
NN_PAD = 10240
RPT = NN_PAD // SC_TILES
DEGW = 16
NBUF = 4

_sc_mesh = plsc.VectorSubcoreMesh(core_axis_name="c", subcore_axis_name="s")
_sc_params = pltpu.CompilerParams(use_tc_tiling_on_sc=False)


def _zero_rows(ref, nrows, ncols):
    zer = jnp.zeros((16,), jnp.float32)

    def body(i, _):
        for k in range(ncols // 16):
            ref[i, pl.ds(k * 16, 16)] = zer
        return 0

    lax.fori_loop(0, nrows, body, 0)


def _fill_ones_rows(ref, nrows, ncols):
    one = jnp.ones((16,), jnp.float32)

    def body(i, _):
        for k in range(ncols // 16):
            ref[i, pl.ds(k * 16, 16)] = one
        return 0

    lax.fori_loop(0, nrows, body, 0)


@functools.partial(
    pl.kernel,
    out_type=jax.ShapeDtypeStruct((SC_CORES, 2, NN_PAD, DEGW), jnp.float32),
    mesh=_sc_mesh,
    compiler_params=_sc_params,
    scratch_types=[
        pltpu.VMEM_SHARED((NN_PAD, DEGW), jnp.float32),
        pltpu.VMEM_SHARED((NN_PAD, DEGW), jnp.float32),
        pltpu.VMEM((CMAXD, CHUNK), jnp.int32),
        pltpu.VMEM((CMAXD, CHUNK), jnp.int32),
        pltpu.VMEM((CHUNK, DEGW), jnp.float32),
        pltpu.VMEM((RPT, DEGW), jnp.float32),
    ],
)
def _deg_kernel(srcm, dstm, out, dout_sh, din_sh, src_v, dst_v, ones_v, zer_v):
    c = lax.axis_index("c")
    s = lax.axis_index("s")
    w = c * SC_TILES + s

    _fill_ones_rows(ones_v, CHUNK, DEGW)
    _zero_rows(zer_v, RPT, DEGW)
    pltpu.sync_copy(zer_v, dout_sh.at[pl.ds(s * RPT, RPT), :])
    pltpu.sync_copy(zer_v, din_sh.at[pl.ds(s * RPT, RPT), :])
    plsc.subcore_barrier()

    lo = w * CMAXD
    pltpu.sync_copy(srcm.at[pl.ds(lo, CMAXD), :], src_v)
    pltpu.sync_copy(dstm.at[pl.ds(lo, CMAXD), :], dst_v)

    def body(j, _):
        pltpu.sync_copy(ones_v, dout_sh.at[src_v.at[j]], add=True)
        pltpu.sync_copy(ones_v, din_sh.at[dst_v.at[j]], add=True)
        return 0

    lax.fori_loop(0, CMAXD, body, 0)
    plsc.subcore_barrier()

    pltpu.sync_copy(dout_sh.at[pl.ds(s * RPT, RPT), :],
                    out.at[c, 0, pl.ds(s * RPT, RPT), :])
    pltpu.sync_copy(din_sh.at[pl.ds(s * RPT, RPT), :],
                    out.at[c, 1, pl.ds(s * RPT, RPT), :])


@functools.partial(
    pl.kernel,
    out_type=jax.ShapeDtypeStruct((SC_CORES, NN_PAD, HALF), jnp.float32),
    mesh=_sc_mesh,
    compiler_params=_sc_params,
    scratch_types=[
        pltpu.VMEM_SHARED((NN_PAD, HALF), jnp.float32),
        pltpu.VMEM((CMAX, CHUNK), jnp.int32),
        pltpu.VMEM((CMAX, CHUNK), jnp.int32),
        pltpu.VMEM((NBUF, CHUNK, HALF), jnp.float32),
        pltpu.VMEM((CHUNK, HALF), jnp.float32),
        pltpu.SemaphoreType.DMA((NBUF,)),
        pltpu.SemaphoreType.DMA((NBUF,)),
    ],
)
def _mp_kernel(xh, srcm, dstm, aggh, agg_sh, src_v, dst_v, rows_v, zer_v,
               gsem, ssem):
    c = lax.axis_index("c")
    s = lax.axis_index("s")

    _zero_rows(zer_v, CHUNK, HALF)
    for k in range(RPT // CHUNK):
        pltpu.sync_copy(zer_v, agg_sh.at[pl.ds(s * RPT + k * CHUNK, CHUNK), :])
    plsc.subcore_barrier()

    lo = s * CMAX
    pltpu.sync_copy(srcm.at[pl.ds(lo, CMAX), :], src_v)
    pltpu.sync_copy(dstm.at[pl.ds(lo, CMAX), :], dst_v)

    def body(q, _):
        for b in range(NBUF):
            jb = q * NBUF + b

            @pl.when(q > 0)
            def _(b=b, jb=jb):
                pltpu.make_async_copy(
                    rows_v.at[b], agg_sh.at[dst_v.at[jb - NBUF]], ssem.at[b]
                ).wait()

            pltpu.async_copy(xh.at[c].at[src_v.at[jb]], rows_v.at[b],
                             gsem.at[b])
        for b in range(NBUF):
            jb = q * NBUF + b
            pltpu.make_async_copy(xh.at[c].at[src_v.at[jb]], rows_v.at[b],
                                  gsem.at[b]).wait()
            pltpu.async_copy(rows_v.at[b], agg_sh.at[dst_v.at[jb]],
                             ssem.at[b], add=True)
        return 0

    lax.fori_loop(0, CMAX // NBUF, body, 0)
    for b in range(NBUF):
        pltpu.make_async_copy(rows_v.at[b],
                              agg_sh.at[dst_v.at[CMAX - NBUF + b]],
                              ssem.at[b]).wait()
    plsc.subcore_barrier()

    pltpu.sync_copy(agg_sh.at[pl.ds(s * RPT, RPT), :],
                    aggh.at[c, pl.ds(s * RPT, RPT), :])


def _embed_body(nf, w, b, ns, h_out, x_out):
    h = jnp.dot(nf[...], w[...], preferred_element_type=jnp.float32) + b[...]
    h_out[...] = h
    xs = h * ns[...]
    x_out[0, pl.ds(0, NN), :] = xs[:, :HALF]
    x_out[1, pl.ds(0, NN), :] = xs[:, HALF:]


def _embed_call(nf, w, b, ns):
    return pl.pallas_call(
        _embed_body,
        out_shape=[
            jax.ShapeDtypeStruct((NN, HH), jnp.float32),
            jax.ShapeDtypeStruct((SC_CORES, NN_PAD, HALF), jnp.float32),
        ],
    )(nf, w, b, ns)


def _layer_body(agg, h_in, nd, nns, w, b, gamma, beta, ns, h_out, x_out):
    a = agg[...]
    aggf = jnp.concatenate([a[0, :NN], a[1, :NN]], axis=1) * nd[...]
    hc = jnp.dot(aggf, w[...], preferred_element_type=jnp.float32) + b[...]
    hc = hc * nns[...]
    mean = jnp.mean(hc, axis=0, keepdims=True)
    cent = hc - mean
    var = jnp.mean(cent * cent, axis=0, keepdims=True)
    hn = cent * lax.rsqrt(var + 1e-5) * gamma[...] + beta[...]
    h = h_in[...] + jnp.maximum(hn, 0.0)
    h_out[...] = h
    xs = h * ns[...]
    x_out[0, pl.ds(0, NN), :] = xs[:, :HALF]
    x_out[1, pl.ds(0, NN), :] = xs[:, HALF:]


def _layer_call(agg, h_in, nd, nns, w, b, gamma, beta, ns):
    return pl.pallas_call(
        _layer_body,
        out_shape=[
            jax.ShapeDtypeStruct((NN, HH), jnp.float32),
            jax.ShapeDtypeStruct((SC_CORES, NN_PAD, HALF), jnp.float32),
        ],
    )(agg, h_in, nd, nns, w, b, gamma, beta, ns)


def _readout_body(h, gid, w0, b0, w1, b1, w2, b2, out):
    iota = lax.broadcasted_iota(jnp.int32, (1, GG), 1)
    onehot = (gid[...] == iota).astype(jnp.float32)
    dn = (((0,), (0,)), ((), ()))
    hsum = lax.dot_general(onehot, h[...], dn,
                           preferred_element_type=jnp.float32)
    counts = lax.dot_general(onehot, jnp.ones((NN, 1), jnp.float32), dn,
                             preferred_element_type=jnp.float32)
    hg = hsum / jnp.maximum(counts, 1.0)
    y = jnp.maximum(jnp.dot(hg, w0[...], preferred_element_type=jnp.float32)
                    + b0[...], 0.0)
    y = jnp.maximum(jnp.dot(y, w1[...], preferred_element_type=jnp.float32)
                    + b1[...], 0.0)
    out[...] = jnp.dot(y, w2[...], preferred_element_type=jnp.float32) + b2[...]


def _readout_call(h, gid, w0, b0, w1, b1, w2, b2):
    return pl.pallas_call(
        _readout_body,
        out_shape=jax.ShapeDtypeStruct((GG, NCLS), jnp.float32),
    )(h, gid, w0, b0, w1, b1, w2, b2)


def kernel(nodes_feat, nodes_num_norm_sqrt, edges_feat, edges_num_norm_sqrt,
           W_embed, b_embed, Ws, bs, gammas, betas,
           W_r0, b_r0, W_r1, b_r1, W_r2, b_r2,
           edge_index, graph_ids):
    pad = jnp.full((2, EPAD - EE), NN, dtype=jnp.int32)
    ei = jnp.concatenate([edge_index, pad], axis=1)
    srcm = ei[0].reshape(NCH, CHUNK)
    dstm = ei[1].reshape(NCH, CHUNK)

    deg = _deg_kernel(srcm, dstm)
    deg_out = deg[0, 0, :NN, 0] + deg[1, 0, :NN, 0]
    deg_in = deg[0, 1, :NN, 0] + deg[1, 1, :NN, 0]
    norm_src = lax.rsqrt(jnp.maximum(deg_out, 1.0)).reshape(NN, 1)
    norm_dst = lax.rsqrt(jnp.maximum(deg_in, 1.0)).reshape(NN, 1)

    h, x = _embed_call(nodes_feat, W_embed, b_embed.reshape(1, HH), norm_src)
    for i in range(LL):
        agg = _mp_kernel(x, srcm, dstm)
        h, x = _layer_call(agg, h, norm_dst, nodes_num_norm_sqrt,
                           Ws[i], bs[i].reshape(1, HH),
                           gammas[i].reshape(1, HH), betas[i].reshape(1, HH),
                           norm_src)

    return _readout_call(h, graph_ids.reshape(NN, 1),
                         W_r0, b_r0.reshape(1, -1),
                         W_r1, b_r1.reshape(1, -1),
                         W_r2, b_r2.reshape(1, -1))
